# baseline (speedup 1.0000x reference)
import functools

import jax
import jax.numpy as jnp
from jax import lax
from jax.experimental import pallas as pl
from jax.experimental.pallas import tpu as pltpu
from jax.experimental.pallas import tpu_sc as plsc

_N = 10000
_E = 160000
_C = 50
_NT = 16
_PER_TILE = _E // _NT
_ITERS = _PER_TILE // _C
_NR = _N // _NT
_F32 = jnp.float32


def _sc_edge_pass(idx4, q_lo, q_hi, kv_lo, kv_hi, keve_lo, keve_hi, zeros):
  mesh = plsc.VectorSubcoreMesh(core_axis_name="c", subcore_axis_name="s")

  @functools.partial(
      pl.kernel, mesh=mesh,
      compiler_params=pltpu.CompilerParams(use_tc_tiling_on_sc=False),
      out_type=jax.ShapeDtypeStruct((2, _N, 128), _F32),
      scratch_types=[
          [pltpu.VMEM((2, _C), jnp.int32) for _ in range(4)],
          [pltpu.VMEM((_C, 64), _F32) for _ in range(2)],
          [pltpu.VMEM((_C, 128), _F32) for _ in range(2)],
          [pltpu.VMEM((_C, 128), _F32) for _ in range(2)],
          [pltpu.VMEM((_C, 128), _F32) for _ in range(2)],
          pltpu.VMEM_SHARED((_N, 128), _F32),
          [pltpu.SemaphoreType.DMA for _ in range(4)],
          [pltpu.SemaphoreType.DMA for _ in range(2)],
          [pltpu.SemaphoreType.DMA for _ in range(2)],
          [pltpu.SemaphoreType.DMA for _ in range(2)],
          [pltpu.SemaphoreType.DMA for _ in range(2)],
      ])
  def kern(idx4_h, qlo_h, qhi_h, kvlo_h, kvhi_h, kelo_h, kehi_h,
           zeros_h, st_out,
           idx_b, q_b, kv_b, e_b, o_b, acc, sx, sq, skv, se, ss):
    cid = lax.axis_index("c")
    sid = lax.axis_index("s")
    tile_base = sid * _PER_TILE
    row0 = sid * _NR

    def run(q_t, kv_t, ke_t):
      def issue_idx(i, r):
        pltpu.async_copy(idx4_h.at[sid, i], idx_b[r], sx[r])

      def wait_idx(i, r):
        pltpu.make_async_copy(idx4_h.at[sid, i], idx_b[r], sx[r]).wait()

      def issue_loads(i, r, b):
        pltpu.async_copy(q_t.at[idx_b[r].at[1]], q_b[b], sq[b])
        pltpu.async_copy(kv_t.at[idx_b[r].at[0]], kv_b[b], skv[b])
        pltpu.async_copy(ke_t.at[pl.ds(tile_base + i * _C, _C)],
                         e_b[b], se[b])

      def wait_loads(i, r, b):
        pltpu.make_async_copy(q_t.at[idx_b[r].at[1]], q_b[b], sq[b]).wait()
        pltpu.make_async_copy(kv_t.at[idx_b[r].at[0]], kv_b[b],
                              skv[b]).wait()
        pltpu.make_async_copy(ke_t.at[pl.ds(tile_base + i * _C, _C)],
                              e_b[b], se[b]).wait()

      def compute(b):
        qb, kvb, eb, ob = q_b[b], kv_b[b], e_b[b], o_b[b]

        @plsc.parallel_loop(0, _C, unroll=4)
        def _(e):
          for j in range(4):
            sl = pl.ds(j * 16, 16)
            s2 = pl.ds(64 + j * 16, 16)
            l = qb[e, sl] * (kvb[e, sl] + eb[e, sl])
            l = jnp.minimum(jnp.maximum(l, -75.0), 75.0)
            p = jnp.exp(l)
            ob[e, sl] = p
            ob[e, s2] = p * (kvb[e, s2] + eb[e, s2])

      pltpu.sync_copy(zeros_h, acc.at[pl.ds(row0, _NR)])
      issue_idx(0, 0)
      issue_idx(1, 1)
      wait_idx(0, 0)
      issue_loads(0, 0, 0)
      plsc.subcore_barrier()

      def body(g, carry):
        for k in range(4):
          i = 4 * g + k
          r = k
          b = k % 2

          @pl.when(i + 1 < _ITERS)
          def _():
            wait_idx(i + 1, (k + 1) % 4)
          wait_loads(i, r, b)

          @pl.when(i + 1 < _ITERS)
          def _():
            issue_loads(i + 1, (k + 1) % 4, (k + 1) % 2)

          @pl.when(i >= 2)
          def _():
            pltpu.make_async_copy(o_b[b], acc.at[idx_b[(k + 2) % 4].at[1]],
                                  ss[b]).wait()

          @pl.when(i + 2 < _ITERS)
          def _():
            issue_idx(i + 2, (k + 2) % 4)

          compute(b)
          pltpu.async_copy(o_b[b], acc.at[idx_b[r].at[1]], ss[b], add=True)
        return carry
      lax.fori_loop(0, _ITERS // 4, body, 0)

      pltpu.make_async_copy(o_b[0], acc.at[idx_b[2].at[1]], ss[0]).wait()
      pltpu.make_async_copy(o_b[1], acc.at[idx_b[3].at[1]], ss[1]).wait()
      plsc.subcore_barrier()
      pltpu.sync_copy(acc.at[pl.ds(row0, _NR)],
                      st_out.at[cid, pl.ds(row0, _NR)])

    @pl.when(cid == 0)
    def _():
      run(qlo_h, kvlo_h, kelo_h)

    @pl.when(cid == 1)
    def _():
      run(qhi_h, kvhi_h, kehi_h)

  return kern(idx4, q_lo, q_hi, kv_lo, kv_hi, keve_lo, keve_hi, zeros)


def _tc_node1(kind, ntype, wt, b):
  bn = 1000

  def body(kind_ref, ntype_ref, w_ref, b_ref, qlo, qhi, kvlo, kvhi):
    x = jnp.concatenate([kind_ref[...], ntype_ref[...]], axis=1)
    y = jnp.dot(x, w_ref[...], preferred_element_type=_F32) + b_ref[...]
    qlo[...] = y[:, 0:64]
    qhi[...] = y[:, 64:128]
    kvlo[...] = jnp.concatenate([y[:, 128:192], y[:, 256:320]], axis=1)
    kvhi[...] = jnp.concatenate([y[:, 192:256], y[:, 320:384]], axis=1)

  return pl.pallas_call(
      body,
      grid=(_N // bn,),
      in_specs=[pl.BlockSpec((bn, 128), lambda i: (i, 0)),
                pl.BlockSpec((bn, 128), lambda i: (i, 0)),
                pl.BlockSpec((256, 384), lambda i: (0, 0)),
                pl.BlockSpec((1, 384), lambda i: (0, 0))],
      out_specs=[pl.BlockSpec((bn, 64), lambda i: (i, 0)),
                 pl.BlockSpec((bn, 64), lambda i: (i, 0)),
                 pl.BlockSpec((bn, 128), lambda i: (i, 0)),
                 pl.BlockSpec((bn, 128), lambda i: (i, 0))],
      out_shape=[jax.ShapeDtypeStruct((_N, 64), _F32),
                 jax.ShapeDtypeStruct((_N, 64), _F32),
                 jax.ShapeDtypeStruct((_N, 128), _F32),
                 jax.ShapeDtypeStruct((_N, 128), _F32)],
  )(kind, ntype, wt, b)


def _tc_edge(edge_attr, wt):
  be = 2000

  def body(x_ref, w_ref, *outs):
    y = jnp.dot(x_ref[...], w_ref[...], preferred_element_type=_F32)
    outs[0][...] = jnp.concatenate([y[:, 0:64], y[:, 128:192]], axis=1)
    outs[1][...] = jnp.concatenate([y[:, 64:128], y[:, 192:256]], axis=1)
    outs[2][...] = jnp.concatenate([y[:, 256:320], y[:, 384:448]], axis=1)
    outs[3][...] = jnp.concatenate([y[:, 320:384], y[:, 448:512]], axis=1)

  return pl.pallas_call(
      body,
      grid=(_E // be,),
      in_specs=[pl.BlockSpec((be, 128), lambda i: (i, 0)),
                pl.BlockSpec((128, 512), lambda i: (0, 0))],
      out_specs=[pl.BlockSpec((be, 128), lambda i: (i, 0))] * 4,
      out_shape=[jax.ShapeDtypeStruct((_E, 128), _F32)] * 4,
  )(edge_attr, wt)


def _split_st(st_ref):
  s = jnp.concatenate([st_ref[0, :, 0:64], st_ref[1, :, 0:64]], axis=1)
  t = jnp.concatenate([st_ref[0, :, 64:128], st_ref[1, :, 64:128]], axis=1)
  return jnp.where(s > 0, t / jnp.where(s > 0, s, 1.0), 0.0)


def _tc_mid(st, kind, ntype, wwt, bw, lng, lnb, wn2t, wh2t, b2):
  bn = 1000

  def body(st_ref, kind_ref, ntype_ref, ww_ref, bw_ref, g_ref, be_ref,
           wn_ref, wh_ref, b2_ref, h_out, qlo, qhi, kvlo, kvhi):
    hn = _split_st(st_ref)
    nc = jnp.concatenate([kind_ref[...], ntype_ref[...]], axis=1)
    x = jnp.concatenate([hn, nc], axis=1)
    hp = jnp.dot(x, ww_ref[...], preferred_element_type=_F32) + bw_ref[...]
    m = jnp.mean(hp, axis=1, keepdims=True)
    var = jnp.mean((hp - m) ** 2, axis=1, keepdims=True)
    h = (hp - m) / jnp.sqrt(var + 1e-5) * g_ref[...] + be_ref[...]
    h_out[...] = h
    y2 = (jnp.dot(nc, wn_ref[...], preferred_element_type=_F32)
          + jnp.dot(h, wh_ref[...], preferred_element_type=_F32) + b2_ref[...])
    qlo[...] = y2[:, 0:64]
    qhi[...] = y2[:, 64:128]
    kvlo[...] = jnp.concatenate([y2[:, 128:192], y2[:, 256:320]], axis=1)
    kvhi[...] = jnp.concatenate([y2[:, 192:256], y2[:, 320:384]], axis=1)

  return pl.pallas_call(
      body,
      grid=(_N // bn,),
      in_specs=[pl.BlockSpec((2, bn, 128), lambda i: (0, i, 0)),
                pl.BlockSpec((bn, 128), lambda i: (i, 0)),
                pl.BlockSpec((bn, 128), lambda i: (i, 0)),
                pl.BlockSpec((384, 128), lambda i: (0, 0)),
                pl.BlockSpec((1, 128), lambda i: (0, 0)),
                pl.BlockSpec((1, 128), lambda i: (0, 0)),
                pl.BlockSpec((1, 128), lambda i: (0, 0)),
                pl.BlockSpec((256, 384), lambda i: (0, 0)),
                pl.BlockSpec((128, 384), lambda i: (0, 0)),
                pl.BlockSpec((1, 384), lambda i: (0, 0))],
      out_specs=[pl.BlockSpec((bn, 128), lambda i: (i, 0)),
                 pl.BlockSpec((bn, 64), lambda i: (i, 0)),
                 pl.BlockSpec((bn, 64), lambda i: (i, 0)),
                 pl.BlockSpec((bn, 128), lambda i: (i, 0)),
                 pl.BlockSpec((bn, 128), lambda i: (i, 0))],
      out_shape=[jax.ShapeDtypeStruct((_N, 128), _F32),
                 jax.ShapeDtypeStruct((_N, 64), _F32),
                 jax.ShapeDtypeStruct((_N, 64), _F32),
                 jax.ShapeDtypeStruct((_N, 128), _F32),
                 jax.ShapeDtypeStruct((_N, 128), _F32)],
  )(st, kind, ntype, wwt, bw, lng, lnb, wn2t, wh2t, b2)


def _tc_final(st, h, kind, ntype, ww2t, bw2, lng, lnb):
  bn = 1000

  def body(st_ref, h_ref, kind_ref, ntype_ref, w_ref, b_ref, g_ref,
           be_ref, h1_out):
    hn = _split_st(st_ref)
    x = jnp.concatenate([hn, h_ref[...], kind_ref[...], ntype_ref[...]],
                        axis=1)
    hp = jnp.dot(x, w_ref[...], preferred_element_type=_F32) + b_ref[...]
    m = jnp.mean(hp, axis=1, keepdims=True)
    var = jnp.mean((hp - m) ** 2, axis=1, keepdims=True)
    h1_out[...] = (hp - m) / jnp.sqrt(var + 1e-5) * g_ref[...] + be_ref[...]

  return pl.pallas_call(
      body,
      grid=(_N // bn,),
      in_specs=[pl.BlockSpec((2, bn, 128), lambda i: (0, i, 0)),
                pl.BlockSpec((bn, 128), lambda i: (i, 0)),
                pl.BlockSpec((bn, 128), lambda i: (i, 0)),
                pl.BlockSpec((bn, 128), lambda i: (i, 0)),
                pl.BlockSpec((512, 128), lambda i: (0, 0)),
                pl.BlockSpec((1, 128), lambda i: (0, 0)),
                pl.BlockSpec((1, 128), lambda i: (0, 0)),
                pl.BlockSpec((1, 128), lambda i: (0, 0))],
      out_specs=pl.BlockSpec((bn, 128), lambda i: (i, 0)),
      out_shape=jax.ShapeDtypeStruct((_N, 128), _F32),
  )(st, h, kind, ntype, ww2t, bw2, lng, lnb)


def _tc_readout(h1, gwt, gb):
  def body(h_ref, gw_ref, gb_ref, out_ref):
    hv = h_ref[...]
    g = jnp.dot(hv, gw_ref[...], preferred_element_type=_F32) + gb_ref[0, 0]
    m = jnp.max(g)
    w = jnp.exp(g - m)
    out_ref[...] = jnp.sum(w * hv, axis=0, keepdims=True) / jnp.sum(w)

  return pl.pallas_call(
      body,
      out_shape=jax.ShapeDtypeStruct((1, 128), _F32),
  )(h1, gwt, gb)


def kernel(kind, ntype, edge_attr, edge_index, WQ, bQ, WK, bK, WV, bV, WW, bW,
           WQ2, bQ2, WK2, bK2, WV2, bV2, WW2, bW2, ln_g, ln_b, gate_w, gate_b):
  idx4 = jnp.stack([edge_index[0].reshape(_NT, _ITERS, _C),
                    edge_index[1].reshape(_NT, _ITERS, _C)], axis=2)

  wt_node1 = jnp.concatenate([WQ, WK[:, :256], WV[:, :256]], axis=0).T
  b_node1 = jnp.concatenate([bQ, bK, bV]).reshape(1, 384)
  wet = jnp.concatenate([WK[:, 256:], WV[:, 256:],
                         WK2[:, 256:384], WV2[:, 256:384]], axis=0).T
  wwt = WW.T
  bw = bW.reshape(1, 128)
  lng = ln_g.reshape(1, 128)
  lnb = ln_b.reshape(1, 128)
  wn2t = jnp.concatenate([WQ2[:, :256], WK2[:, :256], WV2[:, :256]], axis=0).T
  wh2t = jnp.concatenate([WQ2[:, 256:], WK2[:, 384:], WV2[:, 384:]], axis=0).T
  b2 = jnp.concatenate([bQ2, bK2, bV2]).reshape(1, 384)
  ww2t = WW2.T
  bw2 = bW2.reshape(1, 128)
  gwt = gate_w.T
  gb = gate_b.reshape(1, 1)
  zeros = jnp.zeros((_NR, 128), _F32)

  q_lo, q_hi, kv_lo, kv_hi = _tc_node1(kind, ntype, wt_node1, b_node1)
  keve_lo, keve_hi, keve2_lo, keve2_hi = _tc_edge(edge_attr, wet)

  st1 = _sc_edge_pass(idx4, q_lo, q_hi, kv_lo, kv_hi,
                      keve_lo, keve_hi, zeros)
  h, q2_lo, q2_hi, kv2_lo, kv2_hi = _tc_mid(st1, kind, ntype, wwt, bw,
                                            lng, lnb, wn2t, wh2t, b2)

  st2 = _sc_edge_pass(idx4, q2_lo, q2_hi, kv2_lo, kv2_hi,
                      keve2_lo, keve2_hi, zeros)
  h1 = _tc_final(st2, h, kind, ntype, ww2t, bw2, lng, lnb)

  return _tc_readout(h1, gwt, gb)


# speedup vs baseline: 1.0230x; 1.0230x over previous
import functools

import jax
import jax.numpy as jnp
from jax import lax
from jax.experimental import pallas as pl
from jax.experimental.pallas import tpu as pltpu
from jax.experimental.pallas import tpu_sc as plsc

_N = 10000
_E = 160000
_C = 50
_NT = 16
_PER_TILE = _E // _NT
_ITERS = _PER_TILE // _C
_NR = _N // _NT
_F32 = jnp.float32


def _sc_edge_pass(idx4, q_lo, q_hi, kv_lo, kv_hi, keve_lo, keve_hi, zeros):
  mesh = plsc.VectorSubcoreMesh(core_axis_name="c", subcore_axis_name="s")

  @functools.partial(
      pl.kernel, mesh=mesh,
      compiler_params=pltpu.CompilerParams(use_tc_tiling_on_sc=False),
      out_type=jax.ShapeDtypeStruct((2, _N, 128), _F32),
      scratch_types=[
          [pltpu.VMEM((2, _C), jnp.int32) for _ in range(4)],
          [pltpu.VMEM((_C, 64), _F32) for _ in range(2)],
          [pltpu.VMEM((_C, 128), _F32) for _ in range(2)],
          [pltpu.VMEM((_C, 128), _F32) for _ in range(2)],
          [pltpu.VMEM((_C, 128), _F32) for _ in range(2)],
          pltpu.VMEM_SHARED((_N, 128), _F32),
          [pltpu.SemaphoreType.DMA for _ in range(4)],
          [pltpu.SemaphoreType.DMA for _ in range(2)],
          [pltpu.SemaphoreType.DMA for _ in range(2)],
          [pltpu.SemaphoreType.DMA for _ in range(2)],
          [pltpu.SemaphoreType.DMA for _ in range(2)],
      ])
  def kern(idx4_h, qlo_h, qhi_h, kvlo_h, kvhi_h, kelo_h, kehi_h,
           zeros_h, st_out,
           idx_b, q_b, kv_b, e_b, o_b, acc, sx, sq, skv, se, ss):
    cid = lax.axis_index("c")
    sid = lax.axis_index("s")
    tile_base = sid * _PER_TILE
    row0 = sid * _NR

    def run(q_t, kv_t, ke_t):
      def issue_idx(i, r):
        pltpu.async_copy(idx4_h.at[sid, i], idx_b[r], sx[r])

      def wait_idx(i, r):
        pltpu.make_async_copy(idx4_h.at[sid, i], idx_b[r], sx[r]).wait()

      def issue_loads(i, r, b):
        pltpu.async_copy(q_t.at[idx_b[r].at[1]], q_b[b], sq[b])
        pltpu.async_copy(kv_t.at[idx_b[r].at[0]], kv_b[b], skv[b])
        pltpu.async_copy(ke_t.at[pl.ds(tile_base + i * _C, _C)],
                         e_b[b], se[b])

      def wait_loads(i, r, b):
        pltpu.make_async_copy(q_t.at[idx_b[r].at[1]], q_b[b], sq[b]).wait()
        pltpu.make_async_copy(kv_t.at[idx_b[r].at[0]], kv_b[b],
                              skv[b]).wait()
        pltpu.make_async_copy(ke_t.at[pl.ds(tile_base + i * _C, _C)],
                              e_b[b], se[b]).wait()

      def compute(b):
        qb, kvb, eb, ob = q_b[b], kv_b[b], e_b[b], o_b[b]

        @plsc.parallel_loop(0, _C, unroll=2)
        def _(e):
          for j in range(4):
            sl = pl.ds(j * 16, 16)
            s2 = pl.ds(64 + j * 16, 16)
            l = qb[e, sl] * (kvb[e, sl] + eb[e, sl])
            l = jnp.minimum(jnp.maximum(l, -75.0), 75.0)
            p = jnp.exp(l)
            ob[e, sl] = p
            ob[e, s2] = p * (kvb[e, s2] + eb[e, s2])

      pltpu.sync_copy(zeros_h, acc.at[pl.ds(row0, _NR)])
      issue_idx(0, 0)
      issue_idx(1, 1)
      wait_idx(0, 0)
      issue_loads(0, 0, 0)
      plsc.subcore_barrier()

      def body(g, carry):
        for k in range(4):
          i = 4 * g + k
          r = k
          b = k % 2

          @pl.when(i + 1 < _ITERS)
          def _():
            wait_idx(i + 1, (k + 1) % 4)
          wait_loads(i, r, b)

          @pl.when(i + 1 < _ITERS)
          def _():
            issue_loads(i + 1, (k + 1) % 4, (k + 1) % 2)

          @pl.when(i >= 2)
          def _():
            pltpu.make_async_copy(o_b[b], acc.at[idx_b[(k + 2) % 4].at[1]],
                                  ss[b]).wait()

          @pl.when(i + 2 < _ITERS)
          def _():
            issue_idx(i + 2, (k + 2) % 4)

          compute(b)
          pltpu.async_copy(o_b[b], acc.at[idx_b[r].at[1]], ss[b], add=True)
        return carry
      lax.fori_loop(0, _ITERS // 4, body, 0)

      pltpu.make_async_copy(o_b[0], acc.at[idx_b[2].at[1]], ss[0]).wait()
      pltpu.make_async_copy(o_b[1], acc.at[idx_b[3].at[1]], ss[1]).wait()
      plsc.subcore_barrier()
      pltpu.sync_copy(acc.at[pl.ds(row0, _NR)],
                      st_out.at[cid, pl.ds(row0, _NR)])

    @pl.when(cid == 0)
    def _():
      run(qlo_h, kvlo_h, kelo_h)

    @pl.when(cid == 1)
    def _():
      run(qhi_h, kvhi_h, kehi_h)

  return kern(idx4, q_lo, q_hi, kv_lo, kv_hi, keve_lo, keve_hi, zeros)


def _tc_node1(kind, ntype, wt, b):
  bn = 1000

  def body(kind_ref, ntype_ref, w_ref, b_ref, qlo, qhi, kvlo, kvhi):
    x = jnp.concatenate([kind_ref[...], ntype_ref[...]], axis=1)
    y = jnp.dot(x, w_ref[...], preferred_element_type=_F32) + b_ref[...]
    qlo[...] = y[:, 0:64]
    qhi[...] = y[:, 64:128]
    kvlo[...] = jnp.concatenate([y[:, 128:192], y[:, 256:320]], axis=1)
    kvhi[...] = jnp.concatenate([y[:, 192:256], y[:, 320:384]], axis=1)

  return pl.pallas_call(
      body,
      grid=(_N // bn,),
      in_specs=[pl.BlockSpec((bn, 128), lambda i: (i, 0)),
                pl.BlockSpec((bn, 128), lambda i: (i, 0)),
                pl.BlockSpec((256, 384), lambda i: (0, 0)),
                pl.BlockSpec((1, 384), lambda i: (0, 0))],
      out_specs=[pl.BlockSpec((bn, 64), lambda i: (i, 0)),
                 pl.BlockSpec((bn, 64), lambda i: (i, 0)),
                 pl.BlockSpec((bn, 128), lambda i: (i, 0)),
                 pl.BlockSpec((bn, 128), lambda i: (i, 0))],
      out_shape=[jax.ShapeDtypeStruct((_N, 64), _F32),
                 jax.ShapeDtypeStruct((_N, 64), _F32),
                 jax.ShapeDtypeStruct((_N, 128), _F32),
                 jax.ShapeDtypeStruct((_N, 128), _F32)],
  )(kind, ntype, wt, b)


def _tc_edge(edge_attr, wt):
  be = 2000

  def body(x_ref, w_ref, *outs):
    y = jnp.dot(x_ref[...], w_ref[...], preferred_element_type=_F32)
    outs[0][...] = jnp.concatenate([y[:, 0:64], y[:, 128:192]], axis=1)
    outs[1][...] = jnp.concatenate([y[:, 64:128], y[:, 192:256]], axis=1)
    outs[2][...] = jnp.concatenate([y[:, 256:320], y[:, 384:448]], axis=1)
    outs[3][...] = jnp.concatenate([y[:, 320:384], y[:, 448:512]], axis=1)

  return pl.pallas_call(
      body,
      grid=(_E // be,),
      in_specs=[pl.BlockSpec((be, 128), lambda i: (i, 0)),
                pl.BlockSpec((128, 512), lambda i: (0, 0))],
      out_specs=[pl.BlockSpec((be, 128), lambda i: (i, 0))] * 4,
      out_shape=[jax.ShapeDtypeStruct((_E, 128), _F32)] * 4,
  )(edge_attr, wt)


def _split_st(st_ref):
  s = jnp.concatenate([st_ref[0, :, 0:64], st_ref[1, :, 0:64]], axis=1)
  t = jnp.concatenate([st_ref[0, :, 64:128], st_ref[1, :, 64:128]], axis=1)
  return jnp.where(s > 0, t / jnp.where(s > 0, s, 1.0), 0.0)


def _tc_mid(st, kind, ntype, wwt, bw, lng, lnb, wn2t, wh2t, b2):
  bn = 1000

  def body(st_ref, kind_ref, ntype_ref, ww_ref, bw_ref, g_ref, be_ref,
           wn_ref, wh_ref, b2_ref, h_out, qlo, qhi, kvlo, kvhi):
    hn = _split_st(st_ref)
    nc = jnp.concatenate([kind_ref[...], ntype_ref[...]], axis=1)
    x = jnp.concatenate([hn, nc], axis=1)
    hp = jnp.dot(x, ww_ref[...], preferred_element_type=_F32) + bw_ref[...]
    m = jnp.mean(hp, axis=1, keepdims=True)
    var = jnp.mean((hp - m) ** 2, axis=1, keepdims=True)
    h = (hp - m) / jnp.sqrt(var + 1e-5) * g_ref[...] + be_ref[...]
    h_out[...] = h
    y2 = (jnp.dot(nc, wn_ref[...], preferred_element_type=_F32)
          + jnp.dot(h, wh_ref[...], preferred_element_type=_F32) + b2_ref[...])
    qlo[...] = y2[:, 0:64]
    qhi[...] = y2[:, 64:128]
    kvlo[...] = jnp.concatenate([y2[:, 128:192], y2[:, 256:320]], axis=1)
    kvhi[...] = jnp.concatenate([y2[:, 192:256], y2[:, 320:384]], axis=1)

  return pl.pallas_call(
      body,
      grid=(_N // bn,),
      in_specs=[pl.BlockSpec((2, bn, 128), lambda i: (0, i, 0)),
                pl.BlockSpec((bn, 128), lambda i: (i, 0)),
                pl.BlockSpec((bn, 128), lambda i: (i, 0)),
                pl.BlockSpec((384, 128), lambda i: (0, 0)),
                pl.BlockSpec((1, 128), lambda i: (0, 0)),
                pl.BlockSpec((1, 128), lambda i: (0, 0)),
                pl.BlockSpec((1, 128), lambda i: (0, 0)),
                pl.BlockSpec((256, 384), lambda i: (0, 0)),
                pl.BlockSpec((128, 384), lambda i: (0, 0)),
                pl.BlockSpec((1, 384), lambda i: (0, 0))],
      out_specs=[pl.BlockSpec((bn, 128), lambda i: (i, 0)),
                 pl.BlockSpec((bn, 64), lambda i: (i, 0)),
                 pl.BlockSpec((bn, 64), lambda i: (i, 0)),
                 pl.BlockSpec((bn, 128), lambda i: (i, 0)),
                 pl.BlockSpec((bn, 128), lambda i: (i, 0))],
      out_shape=[jax.ShapeDtypeStruct((_N, 128), _F32),
                 jax.ShapeDtypeStruct((_N, 64), _F32),
                 jax.ShapeDtypeStruct((_N, 64), _F32),
                 jax.ShapeDtypeStruct((_N, 128), _F32),
                 jax.ShapeDtypeStruct((_N, 128), _F32)],
  )(st, kind, ntype, wwt, bw, lng, lnb, wn2t, wh2t, b2)


def _tc_final(st, h, kind, ntype, ww2t, bw2, lng, lnb):
  bn = 1000

  def body(st_ref, h_ref, kind_ref, ntype_ref, w_ref, b_ref, g_ref,
           be_ref, h1_out):
    hn = _split_st(st_ref)
    x = jnp.concatenate([hn, h_ref[...], kind_ref[...], ntype_ref[...]],
                        axis=1)
    hp = jnp.dot(x, w_ref[...], preferred_element_type=_F32) + b_ref[...]
    m = jnp.mean(hp, axis=1, keepdims=True)
    var = jnp.mean((hp - m) ** 2, axis=1, keepdims=True)
    h1_out[...] = (hp - m) / jnp.sqrt(var + 1e-5) * g_ref[...] + be_ref[...]

  return pl.pallas_call(
      body,
      grid=(_N // bn,),
      in_specs=[pl.BlockSpec((2, bn, 128), lambda i: (0, i, 0)),
                pl.BlockSpec((bn, 128), lambda i: (i, 0)),
                pl.BlockSpec((bn, 128), lambda i: (i, 0)),
                pl.BlockSpec((bn, 128), lambda i: (i, 0)),
                pl.BlockSpec((512, 128), lambda i: (0, 0)),
                pl.BlockSpec((1, 128), lambda i: (0, 0)),
                pl.BlockSpec((1, 128), lambda i: (0, 0)),
                pl.BlockSpec((1, 128), lambda i: (0, 0))],
      out_specs=pl.BlockSpec((bn, 128), lambda i: (i, 0)),
      out_shape=jax.ShapeDtypeStruct((_N, 128), _F32),
  )(st, h, kind, ntype, ww2t, bw2, lng, lnb)


def _tc_readout(h1, gwt, gb):
  def body(h_ref, gw_ref, gb_ref, out_ref):
    hv = h_ref[...]
    g = jnp.dot(hv, gw_ref[...], preferred_element_type=_F32) + gb_ref[0, 0]
    m = jnp.max(g)
    w = jnp.exp(g - m)
    out_ref[...] = jnp.sum(w * hv, axis=0, keepdims=True) / jnp.sum(w)

  return pl.pallas_call(
      body,
      out_shape=jax.ShapeDtypeStruct((1, 128), _F32),
  )(h1, gwt, gb)


def kernel(kind, ntype, edge_attr, edge_index, WQ, bQ, WK, bK, WV, bV, WW, bW,
           WQ2, bQ2, WK2, bK2, WV2, bV2, WW2, bW2, ln_g, ln_b, gate_w, gate_b):
  idx4 = jnp.stack([edge_index[0].reshape(_NT, _ITERS, _C),
                    edge_index[1].reshape(_NT, _ITERS, _C)], axis=2)

  wt_node1 = jnp.concatenate([WQ, WK[:, :256], WV[:, :256]], axis=0).T
  b_node1 = jnp.concatenate([bQ, bK, bV]).reshape(1, 384)
  wet = jnp.concatenate([WK[:, 256:], WV[:, 256:],
                         WK2[:, 256:384], WV2[:, 256:384]], axis=0).T
  wwt = WW.T
  bw = bW.reshape(1, 128)
  lng = ln_g.reshape(1, 128)
  lnb = ln_b.reshape(1, 128)
  wn2t = jnp.concatenate([WQ2[:, :256], WK2[:, :256], WV2[:, :256]], axis=0).T
  wh2t = jnp.concatenate([WQ2[:, 256:], WK2[:, 384:], WV2[:, 384:]], axis=0).T
  b2 = jnp.concatenate([bQ2, bK2, bV2]).reshape(1, 384)
  ww2t = WW2.T
  bw2 = bW2.reshape(1, 128)
  gwt = gate_w.T
  gb = gate_b.reshape(1, 1)
  zeros = jnp.zeros((_NR, 128), _F32)

  q_lo, q_hi, kv_lo, kv_hi = _tc_node1(kind, ntype, wt_node1, b_node1)
  keve_lo, keve_hi, keve2_lo, keve2_hi = _tc_edge(edge_attr, wet)

  st1 = _sc_edge_pass(idx4, q_lo, q_hi, kv_lo, kv_hi,
                      keve_lo, keve_hi, zeros)
  h, q2_lo, q2_hi, kv2_lo, kv2_hi = _tc_mid(st1, kind, ntype, wwt, bw,
                                            lng, lnb, wn2t, wh2t, b2)

  st2 = _sc_edge_pass(idx4, q2_lo, q2_hi, kv2_lo, kv2_hi,
                      keve2_lo, keve2_hi, zeros)
  h1 = _tc_final(st2, h, kind, ntype, ww2t, bw2, lng, lnb)

  return _tc_readout(h1, gwt, gb)


# speedup vs baseline: 1.0266x; 1.0034x over previous
import functools

import jax
import jax.numpy as jnp
from jax import lax
from jax.experimental import pallas as pl
from jax.experimental.pallas import tpu as pltpu
from jax.experimental.pallas import tpu_sc as plsc

_N = 10000
_E = 160000
_C = 50
_NT = 16
_PER_TILE = _E // _NT
_ITERS = _PER_TILE // _C
_NR = _N // _NT
_F32 = jnp.float32


def _sc_edge_pass(idx4, q_lo, q_hi, kv_lo, kv_hi, keve_lo, keve_hi, zeros):
  mesh = plsc.VectorSubcoreMesh(core_axis_name="c", subcore_axis_name="s")

  @functools.partial(
      pl.kernel, mesh=mesh,
      compiler_params=pltpu.CompilerParams(use_tc_tiling_on_sc=False),
      out_type=jax.ShapeDtypeStruct((2, _N, 128), _F32),
      scratch_types=[
          [pltpu.VMEM((2, _C), jnp.int32) for _ in range(4)],
          [pltpu.VMEM((_C, 64), _F32) for _ in range(2)],
          [pltpu.VMEM((_C, 128), _F32) for _ in range(2)],
          [pltpu.VMEM((_C, 128), _F32) for _ in range(2)],
          [pltpu.VMEM((_C, 128), _F32) for _ in range(2)],
          pltpu.VMEM_SHARED((_N, 128), _F32),
          [pltpu.SemaphoreType.DMA for _ in range(4)],
          [pltpu.SemaphoreType.DMA for _ in range(2)],
          [pltpu.SemaphoreType.DMA for _ in range(2)],
          [pltpu.SemaphoreType.DMA for _ in range(2)],
          [pltpu.SemaphoreType.DMA for _ in range(2)],
      ])
  def kern(idx4_h, qlo_h, qhi_h, kvlo_h, kvhi_h, kelo_h, kehi_h,
           zeros_h, st_out,
           idx_b, q_b, kv_b, e_b, o_b, acc, sx, sq, skv, se, ss):
    cid = lax.axis_index("c")
    sid = lax.axis_index("s")
    tile_base = sid * _PER_TILE
    row0 = sid * _NR

    def run(q_t, kv_t, ke_t):
      def issue_idx(i, r):
        pltpu.async_copy(idx4_h.at[sid, i], idx_b[r], sx[r])

      def wait_idx(i, r):
        pltpu.make_async_copy(idx4_h.at[sid, i], idx_b[r], sx[r]).wait()

      def issue_loads(i, r, b):
        pltpu.async_copy(q_t.at[idx_b[r].at[1]], q_b[b], sq[b])
        pltpu.async_copy(kv_t.at[idx_b[r].at[0]], kv_b[b], skv[b])
        pltpu.async_copy(ke_t.at[pl.ds(tile_base + i * _C, _C)],
                         e_b[b], se[b])

      def wait_loads(i, r, b):
        pltpu.make_async_copy(q_t.at[idx_b[r].at[1]], q_b[b], sq[b]).wait()
        pltpu.make_async_copy(kv_t.at[idx_b[r].at[0]], kv_b[b],
                              skv[b]).wait()
        pltpu.make_async_copy(ke_t.at[pl.ds(tile_base + i * _C, _C)],
                              e_b[b], se[b]).wait()

      def compute(b):
        qb, kvb, eb, ob = q_b[b], kv_b[b], e_b[b], o_b[b]

        @plsc.parallel_loop(0, _C, unroll=2)
        def _(e):
          for j in range(4):
            sl = pl.ds(j * 16, 16)
            s2 = pl.ds(64 + j * 16, 16)
            l = qb[e, sl] * (kvb[e, sl] + eb[e, sl])
            l = jnp.minimum(jnp.maximum(l, -75.0), 75.0)
            p = jnp.exp(l)
            ob[e, sl] = p
            ob[e, s2] = p * (kvb[e, s2] + eb[e, s2])

      pltpu.sync_copy(zeros_h, acc.at[pl.ds(row0, _NR)])
      issue_idx(0, 0)
      issue_idx(1, 1)
      wait_idx(0, 0)
      issue_loads(0, 0, 0)
      plsc.subcore_barrier()

      def body(g, carry):
        for k in range(4):
          i = 4 * g + k
          r = k
          b = k % 2

          @pl.when(i + 1 < _ITERS)
          def _():
            wait_idx(i + 1, (k + 1) % 4)
          wait_loads(i, r, b)

          @pl.when(i + 1 < _ITERS)
          def _():
            issue_loads(i + 1, (k + 1) % 4, (k + 1) % 2)

          @pl.when(i >= 2)
          def _():
            pltpu.make_async_copy(o_b[b], acc.at[idx_b[(k + 2) % 4].at[1]],
                                  ss[b]).wait()

          @pl.when(i + 2 < _ITERS)
          def _():
            issue_idx(i + 2, (k + 2) % 4)

          compute(b)
          pltpu.async_copy(o_b[b], acc.at[idx_b[r].at[1]], ss[b], add=True)
        return carry
      lax.fori_loop(0, _ITERS // 4, body, 0)

      pltpu.make_async_copy(o_b[0], acc.at[idx_b[2].at[1]], ss[0]).wait()
      pltpu.make_async_copy(o_b[1], acc.at[idx_b[3].at[1]], ss[1]).wait()
      plsc.subcore_barrier()
      pltpu.sync_copy(acc.at[pl.ds(row0, _NR)],
                      st_out.at[cid, pl.ds(row0, _NR)])

    @pl.when(cid == 0)
    def _():
      run(qlo_h, kvlo_h, kelo_h)

    @pl.when(cid == 1)
    def _():
      run(qhi_h, kvhi_h, kehi_h)

  return kern(idx4, q_lo, q_hi, kv_lo, kv_hi, keve_lo, keve_hi, zeros)


def _tc_node1(kind, ntype, wt, b):
  bn = 1000

  def body(kind_ref, ntype_ref, w_ref, b_ref, qlo, qhi, kvlo, kvhi):
    x = jnp.concatenate([kind_ref[...], ntype_ref[...]], axis=1)
    y = jnp.dot(x, w_ref[...], preferred_element_type=_F32) + b_ref[...]
    qlo[...] = y[:, 0:64]
    qhi[...] = y[:, 64:128]
    kvlo[...] = jnp.concatenate([y[:, 128:192], y[:, 256:320]], axis=1)
    kvhi[...] = jnp.concatenate([y[:, 192:256], y[:, 320:384]], axis=1)

  return pl.pallas_call(
      body,
      grid=(_N // bn,),
      in_specs=[pl.BlockSpec((bn, 128), lambda i: (i, 0)),
                pl.BlockSpec((bn, 128), lambda i: (i, 0)),
                pl.BlockSpec((256, 384), lambda i: (0, 0)),
                pl.BlockSpec((1, 384), lambda i: (0, 0))],
      out_specs=[pl.BlockSpec((bn, 64), lambda i: (i, 0)),
                 pl.BlockSpec((bn, 64), lambda i: (i, 0)),
                 pl.BlockSpec((bn, 128), lambda i: (i, 0)),
                 pl.BlockSpec((bn, 128), lambda i: (i, 0))],
      out_shape=[jax.ShapeDtypeStruct((_N, 64), _F32),
                 jax.ShapeDtypeStruct((_N, 64), _F32),
                 jax.ShapeDtypeStruct((_N, 128), _F32),
                 jax.ShapeDtypeStruct((_N, 128), _F32)],
  )(kind, ntype, wt, b)


def _tc_edge(edge_attr, wt):
  be = 2000

  def body(x_ref, w_ref, *outs):
    y = jnp.dot(x_ref[...], w_ref[...], preferred_element_type=_F32)
    outs[0][...] = jnp.concatenate([y[:, 0:64], y[:, 128:192]], axis=1)
    outs[1][...] = jnp.concatenate([y[:, 64:128], y[:, 192:256]], axis=1)

  return pl.pallas_call(
      body,
      grid=(_E // be,),
      in_specs=[pl.BlockSpec((be, 128), lambda i: (i, 0)),
                pl.BlockSpec((128, 256), lambda i: (0, 0))],
      out_specs=[pl.BlockSpec((be, 128), lambda i: (i, 0))] * 2,
      out_shape=[jax.ShapeDtypeStruct((_E, 128), _F32)] * 2,
  )(edge_attr, wt)


def _split_st(st_ref):
  s = jnp.concatenate([st_ref[0, :, 0:64], st_ref[1, :, 0:64]], axis=1)
  t = jnp.concatenate([st_ref[0, :, 64:128], st_ref[1, :, 64:128]], axis=1)
  return jnp.where(s > 0, t / jnp.where(s > 0, s, 1.0), 0.0)


def _tc_mid(st, kind, ntype, wwt, bw, lng, lnb, wn2t, wh2t, b2):
  bn = 1000

  def body(st_ref, kind_ref, ntype_ref, ww_ref, bw_ref, g_ref, be_ref,
           wn_ref, wh_ref, b2_ref, h_out, qlo, qhi, kvlo, kvhi):
    hn = _split_st(st_ref)
    nc = jnp.concatenate([kind_ref[...], ntype_ref[...]], axis=1)
    x = jnp.concatenate([hn, nc], axis=1)
    hp = jnp.dot(x, ww_ref[...], preferred_element_type=_F32) + bw_ref[...]
    m = jnp.mean(hp, axis=1, keepdims=True)
    var = jnp.mean((hp - m) ** 2, axis=1, keepdims=True)
    h = (hp - m) / jnp.sqrt(var + 1e-5) * g_ref[...] + be_ref[...]
    h_out[...] = h
    y2 = (jnp.dot(nc, wn_ref[...], preferred_element_type=_F32)
          + jnp.dot(h, wh_ref[...], preferred_element_type=_F32) + b2_ref[...])
    qlo[...] = y2[:, 0:64]
    qhi[...] = y2[:, 64:128]
    kvlo[...] = jnp.concatenate([y2[:, 128:192], y2[:, 256:320]], axis=1)
    kvhi[...] = jnp.concatenate([y2[:, 192:256], y2[:, 320:384]], axis=1)

  return pl.pallas_call(
      body,
      grid=(_N // bn,),
      in_specs=[pl.BlockSpec((2, bn, 128), lambda i: (0, i, 0)),
                pl.BlockSpec((bn, 128), lambda i: (i, 0)),
                pl.BlockSpec((bn, 128), lambda i: (i, 0)),
                pl.BlockSpec((384, 128), lambda i: (0, 0)),
                pl.BlockSpec((1, 128), lambda i: (0, 0)),
                pl.BlockSpec((1, 128), lambda i: (0, 0)),
                pl.BlockSpec((1, 128), lambda i: (0, 0)),
                pl.BlockSpec((256, 384), lambda i: (0, 0)),
                pl.BlockSpec((128, 384), lambda i: (0, 0)),
                pl.BlockSpec((1, 384), lambda i: (0, 0))],
      out_specs=[pl.BlockSpec((bn, 128), lambda i: (i, 0)),
                 pl.BlockSpec((bn, 64), lambda i: (i, 0)),
                 pl.BlockSpec((bn, 64), lambda i: (i, 0)),
                 pl.BlockSpec((bn, 128), lambda i: (i, 0)),
                 pl.BlockSpec((bn, 128), lambda i: (i, 0))],
      out_shape=[jax.ShapeDtypeStruct((_N, 128), _F32),
                 jax.ShapeDtypeStruct((_N, 64), _F32),
                 jax.ShapeDtypeStruct((_N, 64), _F32),
                 jax.ShapeDtypeStruct((_N, 128), _F32),
                 jax.ShapeDtypeStruct((_N, 128), _F32)],
  )(st, kind, ntype, wwt, bw, lng, lnb, wn2t, wh2t, b2)


def _tc_final(st, h, kind, ntype, ww2t, bw2, lng, lnb):
  bn = 1000

  def body(st_ref, h_ref, kind_ref, ntype_ref, w_ref, b_ref, g_ref,
           be_ref, h1_out):
    hn = _split_st(st_ref)
    x = jnp.concatenate([hn, h_ref[...], kind_ref[...], ntype_ref[...]],
                        axis=1)
    hp = jnp.dot(x, w_ref[...], preferred_element_type=_F32) + b_ref[...]
    m = jnp.mean(hp, axis=1, keepdims=True)
    var = jnp.mean((hp - m) ** 2, axis=1, keepdims=True)
    h1_out[...] = (hp - m) / jnp.sqrt(var + 1e-5) * g_ref[...] + be_ref[...]

  return pl.pallas_call(
      body,
      grid=(_N // bn,),
      in_specs=[pl.BlockSpec((2, bn, 128), lambda i: (0, i, 0)),
                pl.BlockSpec((bn, 128), lambda i: (i, 0)),
                pl.BlockSpec((bn, 128), lambda i: (i, 0)),
                pl.BlockSpec((bn, 128), lambda i: (i, 0)),
                pl.BlockSpec((512, 128), lambda i: (0, 0)),
                pl.BlockSpec((1, 128), lambda i: (0, 0)),
                pl.BlockSpec((1, 128), lambda i: (0, 0)),
                pl.BlockSpec((1, 128), lambda i: (0, 0))],
      out_specs=pl.BlockSpec((bn, 128), lambda i: (i, 0)),
      out_shape=jax.ShapeDtypeStruct((_N, 128), _F32),
  )(st, h, kind, ntype, ww2t, bw2, lng, lnb)


def _tc_readout(h1, gwt, gb):
  def body(h_ref, gw_ref, gb_ref, out_ref):
    hv = h_ref[...]
    g = jnp.dot(hv, gw_ref[...], preferred_element_type=_F32) + gb_ref[0, 0]
    m = jnp.max(g)
    w = jnp.exp(g - m)
    out_ref[...] = jnp.sum(w * hv, axis=0, keepdims=True) / jnp.sum(w)

  return pl.pallas_call(
      body,
      out_shape=jax.ShapeDtypeStruct((1, 128), _F32),
  )(h1, gwt, gb)


def kernel(kind, ntype, edge_attr, edge_index, WQ, bQ, WK, bK, WV, bV, WW, bW,
           WQ2, bQ2, WK2, bK2, WV2, bV2, WW2, bW2, ln_g, ln_b, gate_w, gate_b):
  idx4 = jnp.stack([edge_index[0].reshape(_NT, _ITERS, _C),
                    edge_index[1].reshape(_NT, _ITERS, _C)], axis=2)

  wt_node1 = jnp.concatenate([WQ, WK[:, :256], WV[:, :256]], axis=0).T
  b_node1 = jnp.concatenate([bQ, bK, bV]).reshape(1, 384)
  wet1 = jnp.concatenate([WK[:, 256:], WV[:, 256:]], axis=0).T
  wet2 = jnp.concatenate([WK2[:, 256:384], WV2[:, 256:384]], axis=0).T
  wwt = WW.T
  bw = bW.reshape(1, 128)
  lng = ln_g.reshape(1, 128)
  lnb = ln_b.reshape(1, 128)
  wn2t = jnp.concatenate([WQ2[:, :256], WK2[:, :256], WV2[:, :256]], axis=0).T
  wh2t = jnp.concatenate([WQ2[:, 256:], WK2[:, 384:], WV2[:, 384:]], axis=0).T
  b2 = jnp.concatenate([bQ2, bK2, bV2]).reshape(1, 384)
  ww2t = WW2.T
  bw2 = bW2.reshape(1, 128)
  gwt = gate_w.T
  gb = gate_b.reshape(1, 1)
  zeros = jnp.zeros((_NR, 128), _F32)

  q_lo, q_hi, kv_lo, kv_hi = _tc_node1(kind, ntype, wt_node1, b_node1)
  keve_lo, keve_hi = _tc_edge(edge_attr, wet1)

  st1 = _sc_edge_pass(idx4, q_lo, q_hi, kv_lo, kv_hi,
                      keve_lo, keve_hi, zeros)
  keve2_lo, keve2_hi = _tc_edge(edge_attr, wet2)
  h, q2_lo, q2_hi, kv2_lo, kv2_hi = _tc_mid(st1, kind, ntype, wwt, bw,
                                            lng, lnb, wn2t, wh2t, b2)

  st2 = _sc_edge_pass(idx4, q2_lo, q2_hi, kv2_lo, kv2_hi,
                      keve2_lo, keve2_hi, zeros)
  h1 = _tc_final(st2, h, kind, ntype, ww2t, bw2, lng, lnb)

  return _tc_readout(h1, gwt, gb)
